# Initial kernel scaffold; baseline (speedup 1.0000x reference)
import jax
import jax.numpy as jnp
from jax.experimental import pallas as pl


def kernel(venue_id, table, gamma, beta):
    raise NotImplementedError("write your pallas kernel here")


# speedup vs baseline: 6.3959x; 6.3959x over previous
import functools

import jax
import jax.numpy as jnp
from jax import lax
from jax.experimental import pallas as pl
from jax.experimental.pallas import tpu as pltpu
from jax.experimental.pallas import tpu_sc as plsc

EPS = 1e-5


def _norm_body(tab_ref, g_ref, b_ref, out_ref):
    x = tab_ref[...]
    mean = jnp.mean(x, axis=1, keepdims=True)
    xc = x - mean
    var = jnp.mean(xc * xc, axis=1, keepdims=True)
    inv = lax.rsqrt(var + EPS)
    out_ref[...] = xc * inv * g_ref[...] + b_ref[...]


def _normalize_table(table, gamma, beta):
    V, D = table.shape
    R = 2048
    grid = (V + R - 1) // R
    return pl.pallas_call(
        _norm_body,
        grid=(grid,),
        in_specs=[
            pl.BlockSpec((R, D), lambda i: (i, 0)),
            pl.BlockSpec((1, D), lambda i: (0, 0)),
            pl.BlockSpec((1, D), lambda i: (0, 0)),
        ],
        out_specs=pl.BlockSpec((R, D), lambda i: (i, 0)),
        out_shape=jax.ShapeDtypeStruct((V, D), jnp.float32),
    )(table, gamma.reshape(1, D), beta.reshape(1, D))


_S = 128
_NB = 4


@functools.partial(jax.jit, static_argnums=(2, 3))
def _sc_gather(ntable, idx2d, chunk, d):
    info = plsc.get_sparse_core_info()
    nc, ns = info.num_cores, info.num_subcores
    nw = nc * ns
    nchunks_total = idx2d.shape[0]
    nchunk = nchunks_total // nw
    nstream = chunk // _S

    mesh = plsc.VectorSubcoreMesh(core_axis_name="c", subcore_axis_name="s")

    @functools.partial(
        pl.kernel,
        mesh=mesh,
        out_type=jax.ShapeDtypeStruct((nchunks_total, chunk, d), jnp.float32),
        scratch_types=[
            pltpu.VMEM((_NB, chunk), jnp.int32),
            pltpu.VMEM((_NB, chunk, d), jnp.float32),
        ]
        + [pltpu.SemaphoreType.DMA] * (2 * _NB),
        compiler_params=pltpu.CompilerParams(use_tc_tiling_on_sc=False),
    )
    def k(tab_hbm, idx_hbm, out_hbm, idx_v, rows_v, *sems):
        gsems, osems = sems[:_NB], sems[_NB:]
        wid = lax.axis_index("s") * nc + lax.axis_index("c")
        ibase = wid * nchunk

        def idx_rows(i):
            return idx_hbm.at[ibase + i]

        def out_rows(i):
            return out_hbm.at[ibase + i]

        def store_cp(i, b):
            return pltpu.make_async_copy(rows_v.at[b], out_rows(i), osems[b])

        def fire_chunk(i, b):
            pltpu.sync_copy(idx_rows(i), idx_v.at[b])

            def fire(j, c):
                pltpu.async_copy(
                    tab_hbm.at[idx_v.at[b].at[pl.ds(j * _S, _S)]],
                    rows_v.at[b].at[pl.ds(j * _S, _S)],
                    gsems[b],
                )
                return c

            lax.fori_loop(0, nstream, fire, 0)

        def gather_wait(i, b):
            pltpu.make_async_copy(out_rows(i), rows_v.at[b], gsems[b]).wait()

        fire_chunk(0, 0)
        fire_chunk(1, 1)

        def round_body(r, carry):
            for b in range(_NB):
                i = _NB * r + b
                bp = (b + 2) % _NB
                gather_wait(i, b)
                store_cp(i, b).start()

                @pl.when(i + 2 < nchunk)
                def _():
                    @pl.when(i >= 2)
                    def _():
                        store_cp(i - 2, bp).wait()

                    fire_chunk(i + 2, bp)

            return carry

        lax.fori_loop(0, nchunk // _NB, round_body, 0)
        store_cp(nchunk - 2, (nchunk - 2) % _NB).wait()
        store_cp(nchunk - 1, (nchunk - 1) % _NB).wait()

    return k(ntable, idx2d)


def kernel(venue_id, table, gamma, beta):
    B, H = venue_id.shape
    V, D = table.shape
    ntable = _normalize_table(table.astype(jnp.float32), gamma, beta)
    chunk = 1024
    idx2d = venue_id.reshape(-1).astype(jnp.int32).reshape(-1, chunk)
    out = _sc_gather(ntable, idx2d, chunk, D)
    return out.reshape(B, H, D)


# speedup vs baseline: 6.4214x; 1.0040x over previous
import functools

import jax
import jax.numpy as jnp
from jax import lax
from jax.experimental import pallas as pl
from jax.experimental.pallas import tpu as pltpu
from jax.experimental.pallas import tpu_sc as plsc

EPS = 1e-5


def _norm_body(tab_ref, g_ref, b_ref, out_ref):
    x = tab_ref[...]
    mean = jnp.mean(x, axis=1, keepdims=True)
    xc = x - mean
    var = jnp.mean(xc * xc, axis=1, keepdims=True)
    inv = lax.rsqrt(var + EPS)
    out_ref[...] = xc * inv * g_ref[...] + b_ref[...]


def _normalize_table(table, gamma, beta):
    V, D = table.shape
    R = 2048
    grid = (V + R - 1) // R
    return pl.pallas_call(
        _norm_body,
        grid=(grid,),
        in_specs=[
            pl.BlockSpec((R, D), lambda i: (i, 0)),
            pl.BlockSpec((1, D), lambda i: (0, 0)),
            pl.BlockSpec((1, D), lambda i: (0, 0)),
        ],
        out_specs=pl.BlockSpec((R, D), lambda i: (i, 0)),
        out_shape=jax.ShapeDtypeStruct((V, D), jnp.float32),
    )(table, gamma.reshape(1, D), beta.reshape(1, D))


_S = 128
_NB = 4


@functools.partial(jax.jit, static_argnums=(2, 3))
def _sc_gather(ntable, ids, chunk_b, d):
    info = plsc.get_sparse_core_info()
    nc, ns = info.num_cores, info.num_subcores
    nw = nc * ns
    b, h = ids.shape
    per_w = b // nw
    nchunk = per_w // chunk_b
    hpad = 256

    mesh = plsc.VectorSubcoreMesh(core_axis_name="c", subcore_axis_name="s")

    @functools.partial(
        pl.kernel,
        mesh=mesh,
        out_type=jax.ShapeDtypeStruct((b, h, d), jnp.float32),
        scratch_types=[
            pltpu.VMEM((_NB, chunk_b, hpad), jnp.int32),
            pltpu.VMEM((_NB, chunk_b, h, d), jnp.float32),
        ]
        + [pltpu.SemaphoreType.DMA] * (2 * _NB),
        compiler_params=pltpu.CompilerParams(use_tc_tiling_on_sc=False),
    )
    def k(tab_hbm, ids_hbm, out_hbm, idx_v, rows_v, *sems):
        gsems, osems = sems[:_NB], sems[_NB:]
        wid = lax.axis_index("s") * nc + lax.axis_index("c")
        bbase = wid * per_w

        def ids_rows(i):
            return ids_hbm.at[pl.ds(bbase + i * chunk_b, chunk_b)]

        def out_rows(i):
            return out_hbm.at[pl.ds(bbase + i * chunk_b, chunk_b)]

        def store_cp(i, b):
            return pltpu.make_async_copy(rows_v.at[b], out_rows(i), osems[b])

        def fire_chunk(i, b):
            pltpu.sync_copy(ids_rows(i), idx_v.at[b].at[:, pl.ds(0, h)])

            def fire(cb, c):
                pltpu.async_copy(
                    tab_hbm.at[idx_v.at[b].at[cb, pl.ds(0, _S)]],
                    rows_v.at[b].at[cb, pl.ds(0, _S)],
                    gsems[b],
                )
                pltpu.async_copy(
                    tab_hbm.at[idx_v.at[b].at[cb, pl.ds(_S, h - _S)]],
                    rows_v.at[b].at[cb, pl.ds(_S, h - _S)],
                    gsems[b],
                )
                return c

            lax.fori_loop(0, chunk_b, fire, 0)

        def gather_wait(i, b):
            pltpu.make_async_copy(out_rows(i), rows_v.at[b], gsems[b]).wait()

        fire_chunk(0, 0)
        fire_chunk(1, 1)

        def round_body(r, carry):
            for b in range(_NB):
                i = _NB * r + b
                bp = (b + 2) % _NB
                gather_wait(i, b)
                store_cp(i, b).start()

                @pl.when(i + 2 < nchunk)
                def _():
                    @pl.when(i >= 2)
                    def _():
                        store_cp(i - 2, bp).wait()

                    fire_chunk(i + 2, bp)

            return carry

        lax.fori_loop(0, nchunk // _NB, round_body, 0)
        store_cp(nchunk - 2, (nchunk - 2) % _NB).wait()
        store_cp(nchunk - 1, (nchunk - 1) % _NB).wait()

    return k(ntable, ids)


def kernel(venue_id, table, gamma, beta):
    B, H = venue_id.shape
    V, D = table.shape
    ntable = _normalize_table(table.astype(jnp.float32), gamma, beta)
    return _sc_gather(ntable, venue_id.astype(jnp.int32), 8, D)


# speedup vs baseline: 12.8133x; 1.9954x over previous
import functools

import jax
import jax.numpy as jnp
from jax import lax
from jax.experimental import pallas as pl
from jax.experimental.pallas import tpu as pltpu
from jax.experimental.pallas import tpu_sc as plsc

EPS = 1e-5


def _norm_body(tab_ref, g_ref, b_ref, out_ref):
    x = tab_ref[...]
    mean = jnp.mean(x, axis=1, keepdims=True)
    xc = x - mean
    var = jnp.mean(xc * xc, axis=1, keepdims=True)
    inv = lax.rsqrt(var + EPS)
    out_ref[...] = xc * inv * g_ref[...] + b_ref[...]


def _normalize_table(table, gamma, beta):
    V, D = table.shape
    R = 2048
    grid = (V + R - 1) // R
    return pl.pallas_call(
        _norm_body,
        grid=(grid,),
        in_specs=[
            pl.BlockSpec((R, D), lambda i: (i, 0)),
            pl.BlockSpec((1, D), lambda i: (0, 0)),
            pl.BlockSpec((1, D), lambda i: (0, 0)),
        ],
        out_specs=pl.BlockSpec((R, D), lambda i: (i, 0)),
        out_shape=jax.ShapeDtypeStruct((V, D), jnp.float32),
    )(table, gamma.reshape(1, D), beta.reshape(1, D))


_HP = 8
_SB = 128


@functools.partial(jax.jit, static_argnums=(2,))
def _sc_gather_t(ntable, ids_t, d):
    info = plsc.get_sparse_core_info()
    nc, ns = info.num_cores, info.num_subcores
    nw = nc * ns
    h, b = ids_t.shape
    nbj = b // _SB
    nhr = h // _HP
    nitems = (nbj * nhr) // nw
    dblks = d // 8

    mesh = plsc.VectorSubcoreMesh(core_axis_name="c", subcore_axis_name="s")

    @functools.partial(
        pl.kernel,
        mesh=mesh,
        out_type=jax.ShapeDtypeStruct((h, dblks, nbj, 8, _SB), jnp.float32),
        scratch_types=[
            pltpu.VMEM((_HP, _SB), jnp.int32),
            pltpu.VMEM((_HP, _SB), jnp.int32),
            pltpu.VMEM((_HP * _SB, d), jnp.float32),
            pltpu.VMEM((_HP * _SB, d), jnp.float32),
            pltpu.VMEM((_HP, dblks, 1, 8, _SB), jnp.float32),
            pltpu.VMEM((_HP, dblks, 1, 8, _SB), jnp.float32),
        ]
        + [pltpu.SemaphoreType.DMA] * 4,
        compiler_params=pltpu.CompilerParams(
            use_tc_tiling_on_sc=False, needs_layout_passes=False),
    )
    def k(tab_hbm, ids_hbm, out_hbm, i0, i1, g0, g1, t0, t1, gs0, gs1, os0, os1):
        idxs, gbufs, tbufs = (i0, i1), (g0, g1), (t0, t1)
        gsems, osems = (gs0, gs1), (os0, os1)
        wid = lax.axis_index("s") * nc + lax.axis_index("c")
        gbase = wid * nitems
        iota = lax.iota(jnp.int32, 16)

        def coords(i):
            g = gbase + i
            return g // nhr, g % nhr

        def fire(i, bb):
            bj, hr = coords(i)
            pltpu.sync_copy(
                ids_hbm.at[pl.ds(hr * _HP, _HP), pl.ds(bj * _SB, _SB)],
                idxs[bb])
            for s in range(_HP):
                pltpu.async_copy(
                    tab_hbm.at[idxs[bb].at[s]],
                    gbufs[bb].at[pl.ds(s * _SB, _SB)],
                    gsems[bb])

        def gather_wait(bb):
            pltpu.make_async_copy(
                tab_hbm.at[pl.ds(0, _HP * _SB)], gbufs[bb], gsems[bb]).wait()

        def out_slab(i):
            bj, hr = coords(i)
            return out_hbm.at[pl.ds(hr * _HP, _HP), :, pl.ds(bj, 1)]

        def store_cp(i, bb):
            return pltpu.make_async_copy(tbufs[bb], out_slab(i), osems[bb])

        def transpose(bb):
            gb, tb = gbufs[bb], tbufs[bb]
            zv = jnp.zeros((16,), jnp.int32)

            def tbody(t, carry):
                h_ = t // d
                dd = t % d
                dblk = dd // 8
                dsub = dd % 8
                colv = zv + dd
                h16 = zv + h_
                dbv = zv + dblk
                dsv = zv + dsub
                for blg in range(_SB // 16):
                    rowv = h_ * _SB + blg * 16 + iota
                    v = plsc.load_gather(gb, [rowv, colv])
                    plsc.store_scatter(
                        tb, [h16, dbv, zv, dsv, blg * 16 + iota], v)
                return carry

            lax.fori_loop(0, _HP * d, tbody, 0)

        fire(0, 0)

        def round_body(r, carry):
            for bb in range(2):
                i = 2 * r + bb
                gather_wait(bb)

                @pl.when(i + 1 < nitems)
                def _():
                    @pl.when(i >= 1)
                    def _():
                        store_cp(i - 1, 1 - bb).wait()

                    fire(i + 1, 1 - bb)

                transpose(bb)
                store_cp(i, bb).start()
            return carry

        lax.fori_loop(0, nitems // 2, round_body, 0)
        store_cp(nitems - 2, 0).wait()
        store_cp(nitems - 1, 1).wait()

    return k(ntable, ids_t)


def kernel(venue_id, table, gamma, beta):
    B, H = venue_id.shape
    V, D = table.shape
    ntable = _normalize_table(table.astype(jnp.float32), gamma, beta)
    ids_t = venue_id.astype(jnp.int32).T
    out5 = _sc_gather_t(ntable, ids_t, D)
    return jnp.transpose(out5, (2, 4, 0, 1, 3)).reshape(B, H, D)
